# baseline (speedup 1.0000x reference)
import dataclasses
import functools

import jax
import jax.numpy as jnp
from jax import lax
from jax.experimental import pallas as pl
from jax.experimental.pallas import tpu as pltpu
from jax.experimental.pallas import tpu_sc as plsc

_N = 10000
_E = 320000
_D = 128
_B = 64
_R = 2000
_G = _N // _R

_K = 128
_NCHUNK = _E // _K
_NC = 2
_NS = 16
_NW = _NC * _NS
_STRIPE = 632
_STRIPE_LAST = _N - (_NS - 1) * _STRIPE

_PREC = lax.Precision.DEFAULT


def _leaky(x):
    return jnp.where(x > 0, x, x * jnp.float32(0.01))


def _sc_compiler_params():
    cp = pltpu.CompilerParams()
    if "needs_layout_passes" in pltpu.CompilerParams.__dataclass_fields__:
        cp = dataclasses.replace(cp, needs_layout_passes=False)
    return cp


_CPT = _NCHUNK // _NW
_NTAIL = _NCHUNK - _CPT * _NW
_NBUF = 3


def _edge_pass(xw, src, dst, vals):
    mesh = plsc.VectorSubcoreMesh(core_axis_name="c", subcore_axis_name="s")

    @functools.partial(
        pl.kernel,
        out_type=jax.ShapeDtypeStruct((_NC, _N, _D), jnp.float32),
        mesh=mesh,
        compiler_params=_sc_compiler_params(),
        scratch_types=(
            [pltpu.VMEM((_K, _D), jnp.float32)] * _NBUF
            + [pltpu.VMEM((_K,), jnp.int32)] * _NBUF
            + [pltpu.VMEM((_K,), jnp.int32)] * _NBUF
            + [pltpu.VMEM((_K,), jnp.float32)] * _NBUF
            + [pltpu.VMEM((1, _K), jnp.int32)] * _NBUF
            + [pltpu.VMEM_SHARED((_N, _D), jnp.float32)]
            + [pltpu.SemaphoreType.DMA] * _NBUF
            + [pltpu.SemaphoreType.DMA] * _NBUF
            + [pltpu.SemaphoreType.DMA] * _NBUF
            + [pltpu.SemaphoreType.DMA]
        ),
    )
    def kern(xw_hbm, src_hbm, dst_hbm, val_hbm, out_hbm, *scratch):
        rows_bufs = scratch[0:_NBUF]
        src_bufs = scratch[_NBUF:2 * _NBUF]
        dst_bufs = scratch[2 * _NBUF:3 * _NBUF]
        val_bufs = scratch[3 * _NBUF:4 * _NBUF]
        idx_rows = scratch[4 * _NBUF:5 * _NBUF]
        acc_sh = scratch[5 * _NBUF]
        psems = scratch[5 * _NBUF + 1:6 * _NBUF + 1]
        gsems = scratch[6 * _NBUF + 1:7 * _NBUF + 1]
        ssems = scratch[7 * _NBUF + 1:8 * _NBUF + 1]
        zsem = scratch[8 * _NBUF + 1]
        rows_a = rows_bufs[0]

        c = lax.axis_index("c")
        s = lax.axis_index("s")
        wid = c * _NS + s

        g0 = wid * _CPT
        gtail = _CPT * _NW + jnp.minimum(wid, _NTAIL - 1)

        def prefetch_idx(g, src_b, dst_b, val_b, sem):
            off = pl.multiple_of(g * _K, _K)
            pltpu.async_copy(src_hbm.at[pl.ds(off, _K)], src_b, sem)
            pltpu.async_copy(dst_hbm.at[pl.ds(off, _K)], dst_b, sem)
            pltpu.async_copy(val_hbm.at[pl.ds(off, _K)], val_b, sem)

        def wait_idx(rows_ref, sem):
            pltpu.make_async_copy(xw_hbm.at[pl.ds(0, 3)],
                                  rows_ref.at[pl.ds(0, 3)], sem).wait()

        for j in range(_NBUF):
            prefetch_idx(g0 + j, src_bufs[j], dst_bufs[j], val_bufs[j],
                         psems[j])

        zero16 = jnp.zeros((16,), jnp.float32)

        @pl.loop(0, _K)
        def _(i):
            for q in range(_D // 16):
                rows_a[i, pl.ds(q * 16, 16)] = zero16

        base = pl.multiple_of(s * _STRIPE, 8)
        zdescs = [pltpu.async_copy(rows_a, acc_sh.at[pl.ds(base + t * _K, _K)],
                                   zsem)
                  for t in range(4)]

        @pl.when(s < _NS - 1)
        def _():
            pltpu.async_copy(rows_a.at[pl.ds(0, _STRIPE - 512)],
                             acc_sh.at[pl.ds(base + 512, _STRIPE - 512)],
                             zsem).wait()

        @pl.when(s == _NS - 1)
        def _():
            pltpu.async_copy(rows_a.at[pl.ds(0, _STRIPE_LAST - 512)],
                             acc_sh.at[pl.ds(base + 512, _STRIPE_LAST - 512)],
                             zsem).wait()

        for d in zdescs:
            d.wait()
        plsc.subcore_barrier()

        def build_idx_row(i_ref, dst_b):
            for q in range(_K // 16):
                i_ref[0, pl.ds(q * 16, 16)] = dst_b[pl.ds(q * 16, 16)]

        def scale_rows(rows_ref, val_b):
            @plsc.parallel_loop(0, _K, unroll=8)
            def _(r):
                vb = plsc.load_gather(val_b, (jnp.full((16,), r, jnp.int32),))
                for q in range(_D // 16):
                    rows_ref[r, pl.ds(q * 16, 16)] = (
                        rows_ref[r, pl.ds(q * 16, 16)] * vb)

        def gather_chunk(rows_ref, src_b, sem):
            return pltpu.async_copy(xw_hbm.at[src_b], rows_ref, sem)

        def scatter_chunk(rows_ref, i_ref, sem):
            return pltpu.async_copy(rows_ref, acc_sh.at[i_ref.at[0]], sem,
                                    add=True)

        def wait_scatter(rows_ref, i_ref, sem):
            pltpu.make_async_copy(rows_ref, acc_sh.at[i_ref.at[0]], sem,
                                  ).wait()

        @pl.loop(0, _CPT // _NBUF)
        def _(t):
            gds = []
            for j in range(_NBUF):
                @pl.when(t > 0)
                def _(j=j):
                    wait_scatter(rows_bufs[j], idx_rows[j], ssems[j])

                wait_idx(rows_bufs[j], psems[j])
                build_idx_row(idx_rows[j], dst_bufs[j])
                gds.append(gather_chunk(rows_bufs[j], src_bufs[j], gsems[j]))

            for j in range(_NBUF):
                gds[j].wait()
                scale_rows(rows_bufs[j], val_bufs[j])
                scatter_chunk(rows_bufs[j], idx_rows[j], ssems[j])
                g_next = g0 + _NBUF * t + _NBUF + j
                prefetch_idx(jnp.where(g_next >= g0 + _CPT, gtail, g_next),
                             src_bufs[j], dst_bufs[j], val_bufs[j], psems[j])

        for j in range(_NBUF):
            wait_scatter(rows_bufs[j], idx_rows[j], ssems[j])
            wait_idx(rows_bufs[j], psems[j])

        @pl.when(wid < _NTAIL)
        def _():
            build_idx_row(idx_rows[0], dst_bufs[0])
            gather_chunk(rows_bufs[0], src_bufs[0], gsems[0]).wait()
            scale_rows(rows_bufs[0], val_bufs[0])
            scatter_chunk(rows_bufs[0], idx_rows[0], ssems[0])
            wait_scatter(rows_bufs[0], idx_rows[0], ssems[0])

        plsc.subcore_barrier()

        wdescs = [pltpu.async_copy(acc_sh.at[pl.ds(base + t * _K, _K)],
                                   out_hbm.at[c, pl.ds(base + t * _K, _K)],
                                   zsem)
                  for t in range(4)]

        @pl.when(s < _NS - 1)
        def _():
            pltpu.async_copy(acc_sh.at[pl.ds(base + 512, _STRIPE - 512)],
                             out_hbm.at[c, pl.ds(base + 512, _STRIPE - 512)],
                             zsem).wait()

        @pl.when(s == _NS - 1)
        def _():
            pltpu.async_copy(acc_sh.at[pl.ds(base + 512, _STRIPE_LAST - 512)],
                             out_hbm.at[c, pl.ds(base + 512, _STRIPE_LAST - 512)],
                             zsem).wait()

        for d in wdescs:
            d.wait()

    return kern(xw, src, dst, vals)


def _root_mask(ridx_ref, i):
    rows = lax.broadcasted_iota(jnp.int32, (_B, _R), 1) + i * _R
    return (ridx_ref[...] == rows).astype(jnp.float32)


def _accum(ref, i, part):
    @pl.when(i == 0)
    def _():
        ref[...] = part

    @pl.when(i != 0)
    def _():
        ref[...] = ref[...] + part


def _tc_stage1(x, W1, ridx_col):
    def body(x_ref, w_ref, r_ref, xw_ref, xr_ref):
        i = pl.program_id(0)
        xb = x_ref[...]
        xw_ref[...] = lax.dot(xb, w_ref[...], precision=_PREC,
                              preferred_element_type=jnp.float32)
        part = lax.dot(_root_mask(r_ref, i), xb, precision=_PREC,
                       preferred_element_type=jnp.float32)
        _accum(xr_ref, i, part)

    return pl.pallas_call(
        body,
        grid=(_G,),
        in_specs=[
            pl.BlockSpec((_R, _D), lambda i: (i, 0)),
            pl.BlockSpec((_D, _D), lambda i: (0, 0)),
            pl.BlockSpec((_B, 1), lambda i: (0, 0)),
        ],
        out_specs=[
            pl.BlockSpec((_R, _D), lambda i: (i, 0)),
            pl.BlockSpec((_B, _D), lambda i: (0, 0)),
        ],
        out_shape=[
            jax.ShapeDtypeStruct((_N, _D), jnp.float32),
            jax.ShapeDtypeStruct((_B, _D), jnp.float32),
        ],
    )(x, W1, ridx_col)


def _tc_stage4(agg1, b1r, W2t, W2b, xroot, batch_col, ridx_col):
    def body(p0_ref, p1_ref, b_ref, wt_ref, wb_ref, xr_ref, bt_ref, r_ref,
             xw2_ref, ar_ref):
        i = pl.program_id(0)
        f1 = p0_ref[0] + p1_ref[0] + b_ref[...]
        g = lax.dot(_leaky(xr_ref[...]), wb_ref[...], precision=_PREC,
                    preferred_element_type=jnp.float32)
        oh = (bt_ref[...] ==
              lax.broadcasted_iota(jnp.int32, (_R, _B), 1)).astype(jnp.float32)
        xw2_ref[...] = (
            lax.dot(_leaky(f1), wt_ref[...], precision=_PREC,
                    preferred_element_type=jnp.float32)
            + lax.dot(oh, g, precision=_PREC, preferred_element_type=jnp.float32))
        part = lax.dot(_root_mask(r_ref, i), f1, precision=_PREC,
                       preferred_element_type=jnp.float32)
        _accum(ar_ref, i, part)

    return pl.pallas_call(
        body,
        grid=(_G,),
        in_specs=[
            pl.BlockSpec((1, _R, _D), lambda i: (0, i, 0)),
            pl.BlockSpec((1, _R, _D), lambda i: (1, i, 0)),
            pl.BlockSpec((1, _D), lambda i: (0, 0)),
            pl.BlockSpec((_D, _D), lambda i: (0, 0)),
            pl.BlockSpec((_D, _D), lambda i: (0, 0)),
            pl.BlockSpec((_B, _D), lambda i: (0, 0)),
            pl.BlockSpec((_R, 1), lambda i: (i, 0)),
            pl.BlockSpec((_B, 1), lambda i: (0, 0)),
        ],
        out_specs=[
            pl.BlockSpec((_R, _D), lambda i: (i, 0)),
            pl.BlockSpec((_B, _D), lambda i: (0, 0)),
        ],
        out_shape=[
            jax.ShapeDtypeStruct((_N, _D), jnp.float32),
            jax.ShapeDtypeStruct((_B, _D), jnp.float32),
        ],
    )(agg1, agg1, b1r, W2t, W2b, xroot, batch_col, ridx_col)


def _tc_stage6(agg2, b2r, Wlt, Wlb, a1root, b1r, batch_col, blr):
    def body(q0_ref, q1_ref, b2_ref, wt_ref, wb_ref, ar_ref, b1_ref, bt_ref,
             bl_ref, out_ref):
        f2 = _leaky(q0_ref[0] + q1_ref[0] + b2_ref[...])
        rW = lax.dot(ar_ref[...] + b1_ref[...], wb_ref[...], precision=_PREC,
                     preferred_element_type=jnp.float32)
        oh = (bt_ref[...] ==
              lax.broadcasted_iota(jnp.int32, (_R, _B), 1)).astype(jnp.float32)
        out_ref[...] = _leaky(
            lax.dot(f2, wt_ref[...], precision=_PREC,
                    preferred_element_type=jnp.float32)
            + lax.dot(oh, rW, precision=_PREC, preferred_element_type=jnp.float32)
            + bl_ref[...])

    return pl.pallas_call(
        body,
        grid=(_G,),
        in_specs=[
            pl.BlockSpec((1, _R, _D), lambda i: (0, i, 0)),
            pl.BlockSpec((1, _R, _D), lambda i: (1, i, 0)),
            pl.BlockSpec((1, _D), lambda i: (0, 0)),
            pl.BlockSpec((_D, _D), lambda i: (0, 0)),
            pl.BlockSpec((_D, _D), lambda i: (0, 0)),
            pl.BlockSpec((_B, _D), lambda i: (0, 0)),
            pl.BlockSpec((1, _D), lambda i: (0, 0)),
            pl.BlockSpec((_R, 1), lambda i: (i, 0)),
            pl.BlockSpec((1, _D), lambda i: (0, 0)),
        ],
        out_specs=pl.BlockSpec((_R, _D), lambda i: (i, 0)),
        out_shape=jax.ShapeDtypeStruct((_N, _D), jnp.float32),
    )(agg2, agg2, b2r, Wlt, Wlb, a1root, b1r, batch_col, blr)


def kernel(features, adjs, values, root_idx, propagation_node_num,
           propagation_edge_num, batch, W1, b1, W2, b2, Wl, bl):
    src = adjs[0]
    dst = adjs[1]
    ridx_col = root_idx.reshape(_B, 1)
    batch_col = batch.reshape(_N, 1)
    b1r = b1.reshape(1, _D)
    b2r = b2.reshape(1, _D)
    blr = bl.reshape(1, _D)
    W2t, W2b = W2[:_D], W2[_D:]
    Wlt, Wlb = Wl[:_D], Wl[_D:]

    xw1, xroot = _tc_stage1(features, W1, ridx_col)
    agg1 = _edge_pass(xw1, src, dst, values)
    xw2, a1root = _tc_stage4(agg1, b1r, W2t, W2b, xroot, batch_col, ridx_col)
    agg2 = _edge_pass(xw2, src, dst, values)
    return _tc_stage6(agg2, b2r, Wlt, Wlb, a1root, b1r, batch_col, blr)


# speedup vs baseline: 1.0343x; 1.0343x over previous
import dataclasses
import functools

import jax
import jax.numpy as jnp
from jax import lax
from jax.experimental import pallas as pl
from jax.experimental.pallas import tpu as pltpu
from jax.experimental.pallas import tpu_sc as plsc

_N = 10000
_E = 320000
_D = 128
_B = 64
_R = 2000
_G = _N // _R

_K = 128
_NCHUNK = _E // _K
_NC = 2
_NS = 16
_NW = _NC * _NS
_STRIPE = 632
_STRIPE_LAST = _N - (_NS - 1) * _STRIPE

_PREC = lax.Precision.DEFAULT


def _leaky(x):
    return jnp.where(x > 0, x, x * jnp.float32(0.01))


def _sc_compiler_params():
    cp = pltpu.CompilerParams()
    if "needs_layout_passes" in pltpu.CompilerParams.__dataclass_fields__:
        cp = dataclasses.replace(cp, needs_layout_passes=False)
    return cp


_CPT = _NCHUNK // _NW
_NTAIL = _NCHUNK - _CPT * _NW
_NBUF = 3


def _edge_pass(xw, adjs_flat, vals):
    mesh = plsc.VectorSubcoreMesh(core_axis_name="c", subcore_axis_name="s")

    @functools.partial(
        pl.kernel,
        out_type=jax.ShapeDtypeStruct((_NC, _N, _D), jnp.float32),
        mesh=mesh,
        compiler_params=_sc_compiler_params(),
        scratch_types=(
            [pltpu.VMEM((_K, _D), jnp.float32)] * _NBUF
            + [pltpu.VMEM((_K,), jnp.int32)] * _NBUF
            + [pltpu.VMEM((_K,), jnp.int32)] * _NBUF
            + [pltpu.VMEM((_K,), jnp.float32)] * _NBUF
            + [pltpu.VMEM((1, _K), jnp.int32)] * _NBUF
            + [pltpu.VMEM_SHARED((_N, _D), jnp.float32)]
            + [pltpu.SemaphoreType.DMA] * _NBUF
            + [pltpu.SemaphoreType.DMA] * _NBUF
            + [pltpu.SemaphoreType.DMA] * _NBUF
            + [pltpu.SemaphoreType.DMA]
        ),
    )
    def kern(xw_hbm, adj_hbm, val_hbm, out_hbm, *scratch):
        rows_bufs = scratch[0:_NBUF]
        src_bufs = scratch[_NBUF:2 * _NBUF]
        dst_bufs = scratch[2 * _NBUF:3 * _NBUF]
        val_bufs = scratch[3 * _NBUF:4 * _NBUF]
        idx_rows = scratch[4 * _NBUF:5 * _NBUF]
        acc_sh = scratch[5 * _NBUF]
        psems = scratch[5 * _NBUF + 1:6 * _NBUF + 1]
        gsems = scratch[6 * _NBUF + 1:7 * _NBUF + 1]
        ssems = scratch[7 * _NBUF + 1:8 * _NBUF + 1]
        zsem = scratch[8 * _NBUF + 1]
        rows_a = rows_bufs[0]

        c = lax.axis_index("c")
        s = lax.axis_index("s")
        wid = c * _NS + s

        g0 = wid * _CPT
        gtail = _CPT * _NW + jnp.minimum(wid, _NTAIL - 1)

        def prefetch_idx(g, src_b, dst_b, val_b, sem):
            off = pl.multiple_of(g * _K, _K)
            pltpu.async_copy(adj_hbm.at[pl.ds(off, _K)], src_b, sem)
            pltpu.async_copy(adj_hbm.at[pl.ds(_E + off, _K)], dst_b, sem)
            pltpu.async_copy(val_hbm.at[pl.ds(off, _K)], val_b, sem)

        def wait_idx(rows_ref, sem):
            pltpu.make_async_copy(xw_hbm.at[pl.ds(0, 3)],
                                  rows_ref.at[pl.ds(0, 3)], sem).wait()

        for j in range(_NBUF):
            prefetch_idx(g0 + j, src_bufs[j], dst_bufs[j], val_bufs[j],
                         psems[j])

        zero16 = jnp.zeros((16,), jnp.float32)

        @pl.loop(0, _K)
        def _(i):
            for q in range(_D // 16):
                rows_a[i, pl.ds(q * 16, 16)] = zero16

        base = pl.multiple_of(s * _STRIPE, 8)
        zdescs = [pltpu.async_copy(rows_a, acc_sh.at[pl.ds(base + t * _K, _K)],
                                   zsem)
                  for t in range(4)]

        @pl.when(s < _NS - 1)
        def _():
            pltpu.async_copy(rows_a.at[pl.ds(0, _STRIPE - 512)],
                             acc_sh.at[pl.ds(base + 512, _STRIPE - 512)],
                             zsem).wait()

        @pl.when(s == _NS - 1)
        def _():
            pltpu.async_copy(rows_a.at[pl.ds(0, _STRIPE_LAST - 512)],
                             acc_sh.at[pl.ds(base + 512, _STRIPE_LAST - 512)],
                             zsem).wait()

        for d in zdescs:
            d.wait()
        plsc.subcore_barrier()

        def build_idx_row(i_ref, dst_b):
            for q in range(_K // 16):
                i_ref[0, pl.ds(q * 16, 16)] = dst_b[pl.ds(q * 16, 16)]

        def scale_rows(rows_ref, val_b):
            @plsc.parallel_loop(0, _K, unroll=8)
            def _(r):
                vb = plsc.load_gather(val_b, (jnp.full((16,), r, jnp.int32),))
                for q in range(_D // 16):
                    rows_ref[r, pl.ds(q * 16, 16)] = (
                        rows_ref[r, pl.ds(q * 16, 16)] * vb)

        def gather_chunk(rows_ref, src_b, sem):
            return pltpu.async_copy(xw_hbm.at[src_b], rows_ref, sem)

        def scatter_chunk(rows_ref, i_ref, sem):
            return pltpu.async_copy(rows_ref, acc_sh.at[i_ref.at[0]], sem,
                                    add=True)

        def wait_scatter(rows_ref, i_ref, sem):
            pltpu.make_async_copy(rows_ref, acc_sh.at[i_ref.at[0]], sem,
                                  ).wait()

        @pl.loop(0, _CPT // _NBUF)
        def _(t):
            gds = []
            for j in range(_NBUF):
                @pl.when(t > 0)
                def _(j=j):
                    wait_scatter(rows_bufs[j], idx_rows[j], ssems[j])

                wait_idx(rows_bufs[j], psems[j])
                build_idx_row(idx_rows[j], dst_bufs[j])
                gds.append(gather_chunk(rows_bufs[j], src_bufs[j], gsems[j]))

            for j in range(_NBUF):
                gds[j].wait()
                scale_rows(rows_bufs[j], val_bufs[j])
                scatter_chunk(rows_bufs[j], idx_rows[j], ssems[j])
                g_next = g0 + _NBUF * t + _NBUF + j
                prefetch_idx(jnp.where(g_next >= g0 + _CPT, gtail, g_next),
                             src_bufs[j], dst_bufs[j], val_bufs[j], psems[j])

        for j in range(_NBUF):
            wait_scatter(rows_bufs[j], idx_rows[j], ssems[j])
            wait_idx(rows_bufs[j], psems[j])

        @pl.when(wid < _NTAIL)
        def _():
            build_idx_row(idx_rows[0], dst_bufs[0])
            gather_chunk(rows_bufs[0], src_bufs[0], gsems[0]).wait()
            scale_rows(rows_bufs[0], val_bufs[0])
            scatter_chunk(rows_bufs[0], idx_rows[0], ssems[0])
            wait_scatter(rows_bufs[0], idx_rows[0], ssems[0])

        plsc.subcore_barrier()

        wdescs = [pltpu.async_copy(acc_sh.at[pl.ds(base + t * _K, _K)],
                                   out_hbm.at[c, pl.ds(base + t * _K, _K)],
                                   zsem)
                  for t in range(4)]

        @pl.when(s < _NS - 1)
        def _():
            pltpu.async_copy(acc_sh.at[pl.ds(base + 512, _STRIPE - 512)],
                             out_hbm.at[c, pl.ds(base + 512, _STRIPE - 512)],
                             zsem).wait()

        @pl.when(s == _NS - 1)
        def _():
            pltpu.async_copy(acc_sh.at[pl.ds(base + 512, _STRIPE_LAST - 512)],
                             out_hbm.at[c, pl.ds(base + 512, _STRIPE_LAST - 512)],
                             zsem).wait()

        for d in wdescs:
            d.wait()

    return kern(xw, adjs_flat, vals)


def _root_mask(ridx_ref, i):
    rows = lax.broadcasted_iota(jnp.int32, (_B, _R), 1) + i * _R
    return (ridx_ref[...] == rows).astype(jnp.float32)


def _accum(ref, i, part):
    @pl.when(i == 0)
    def _():
        ref[...] = part

    @pl.when(i != 0)
    def _():
        ref[...] = ref[...] + part


def _tc_stage1(x, W1, ridx_col):
    def body(x_ref, w_ref, r_ref, xw_ref, xr_ref):
        i = pl.program_id(0)
        xb = x_ref[...]
        xw_ref[...] = lax.dot(xb, w_ref[...], precision=_PREC,
                              preferred_element_type=jnp.float32)
        part = lax.dot(_root_mask(r_ref, i), xb, precision=_PREC,
                       preferred_element_type=jnp.float32)
        _accum(xr_ref, i, part)

    return pl.pallas_call(
        body,
        grid=(_G,),
        in_specs=[
            pl.BlockSpec((_R, _D), lambda i: (i, 0)),
            pl.BlockSpec((_D, _D), lambda i: (0, 0)),
            pl.BlockSpec((_B, 1), lambda i: (0, 0)),
        ],
        out_specs=[
            pl.BlockSpec((_R, _D), lambda i: (i, 0)),
            pl.BlockSpec((_B, _D), lambda i: (0, 0)),
        ],
        out_shape=[
            jax.ShapeDtypeStruct((_N, _D), jnp.float32),
            jax.ShapeDtypeStruct((_B, _D), jnp.float32),
        ],
    )(x, W1, ridx_col)


def _tc_stage4(agg1, b1r, W2t, W2b, xroot, batch_col, ridx_col):
    def body(p0_ref, p1_ref, b_ref, wt_ref, wb_ref, xr_ref, bt_ref, r_ref,
             xw2_ref, ar_ref):
        i = pl.program_id(0)
        f1 = p0_ref[0] + p1_ref[0] + b_ref[...]
        g = lax.dot(_leaky(xr_ref[...]), wb_ref[...], precision=_PREC,
                    preferred_element_type=jnp.float32)
        oh = (bt_ref[...] ==
              lax.broadcasted_iota(jnp.int32, (_R, _B), 1)).astype(jnp.float32)
        xw2_ref[...] = (
            lax.dot(_leaky(f1), wt_ref[...], precision=_PREC,
                    preferred_element_type=jnp.float32)
            + lax.dot(oh, g, precision=_PREC, preferred_element_type=jnp.float32))
        part = lax.dot(_root_mask(r_ref, i), f1, precision=_PREC,
                       preferred_element_type=jnp.float32)
        _accum(ar_ref, i, part)

    return pl.pallas_call(
        body,
        grid=(_G,),
        in_specs=[
            pl.BlockSpec((1, _R, _D), lambda i: (0, i, 0)),
            pl.BlockSpec((1, _R, _D), lambda i: (1, i, 0)),
            pl.BlockSpec((1, _D), lambda i: (0, 0)),
            pl.BlockSpec((_D, _D), lambda i: (0, 0)),
            pl.BlockSpec((_D, _D), lambda i: (0, 0)),
            pl.BlockSpec((_B, _D), lambda i: (0, 0)),
            pl.BlockSpec((_R, 1), lambda i: (i, 0)),
            pl.BlockSpec((_B, 1), lambda i: (0, 0)),
        ],
        out_specs=[
            pl.BlockSpec((_R, _D), lambda i: (i, 0)),
            pl.BlockSpec((_B, _D), lambda i: (0, 0)),
        ],
        out_shape=[
            jax.ShapeDtypeStruct((_N, _D), jnp.float32),
            jax.ShapeDtypeStruct((_B, _D), jnp.float32),
        ],
    )(agg1, agg1, b1r, W2t, W2b, xroot, batch_col, ridx_col)


def _tc_stage6(agg2, b2r, Wlt, Wlb, a1root, b1r, batch_col, blr):
    def body(q0_ref, q1_ref, b2_ref, wt_ref, wb_ref, ar_ref, b1_ref, bt_ref,
             bl_ref, out_ref):
        f2 = _leaky(q0_ref[0] + q1_ref[0] + b2_ref[...])
        rW = lax.dot(ar_ref[...] + b1_ref[...], wb_ref[...], precision=_PREC,
                     preferred_element_type=jnp.float32)
        oh = (bt_ref[...] ==
              lax.broadcasted_iota(jnp.int32, (_R, _B), 1)).astype(jnp.float32)
        out_ref[...] = _leaky(
            lax.dot(f2, wt_ref[...], precision=_PREC,
                    preferred_element_type=jnp.float32)
            + lax.dot(oh, rW, precision=_PREC, preferred_element_type=jnp.float32)
            + bl_ref[...])

    return pl.pallas_call(
        body,
        grid=(_G,),
        in_specs=[
            pl.BlockSpec((1, _R, _D), lambda i: (0, i, 0)),
            pl.BlockSpec((1, _R, _D), lambda i: (1, i, 0)),
            pl.BlockSpec((1, _D), lambda i: (0, 0)),
            pl.BlockSpec((_D, _D), lambda i: (0, 0)),
            pl.BlockSpec((_D, _D), lambda i: (0, 0)),
            pl.BlockSpec((_B, _D), lambda i: (0, 0)),
            pl.BlockSpec((1, _D), lambda i: (0, 0)),
            pl.BlockSpec((_R, 1), lambda i: (i, 0)),
            pl.BlockSpec((1, _D), lambda i: (0, 0)),
        ],
        out_specs=pl.BlockSpec((_R, _D), lambda i: (i, 0)),
        out_shape=jax.ShapeDtypeStruct((_N, _D), jnp.float32),
    )(agg2, agg2, b2r, Wlt, Wlb, a1root, b1r, batch_col, blr)


def kernel(features, adjs, values, root_idx, propagation_node_num,
           propagation_edge_num, batch, W1, b1, W2, b2, Wl, bl):
    adjs_flat = adjs.reshape(2 * _E)
    ridx_col = root_idx.reshape(_B, 1)
    batch_col = batch.reshape(_N, 1)
    b1r = b1.reshape(1, _D)
    b2r = b2.reshape(1, _D)
    blr = bl.reshape(1, _D)
    W2t, W2b = W2[:_D], W2[_D:]
    Wlt, Wlb = Wl[:_D], Wl[_D:]

    xw1, xroot = _tc_stage1(features, W1, ridx_col)
    agg1 = _edge_pass(xw1, adjs_flat, values)
    xw2, a1root = _tc_stage4(agg1, b1r, W2t, W2b, xroot, batch_col, ridx_col)
    agg2 = _edge_pass(xw2, adjs_flat, values)
    return _tc_stage6(agg2, b2r, Wlt, Wlb, a1root, b1r, batch_col, blr)
